# baseline (speedup 1.0000x reference)
import functools

import jax
import jax.numpy as jnp
from jax import lax
from jax.experimental import pallas as pl
from jax.experimental.pallas import tpu as pltpu
from jax.experimental.pallas import tpu_sc as plsc

N = 10000
E = 320000
D = 128
DE = 16
G = 16

NC = 2
NS = 16
NW = NC * NS
CHUNK = 64
TOTAL_CHUNKS = E // CHUNK
CHUNKS_LO = TOTAL_CHUNKS // NW
CHUNKS_REM = TOTAL_CHUNKS % NW
NP = 10112
ROWS_PER_TILE = NP // NS
ZROWS = CHUNK
ZTAIL = ROWS_PER_TILE - (ROWS_PER_TILE // ZROWS) * ZROWS


def _sc_msg_body(x_hbm, ea_hbm, src_hbm, dst_hbm, out_hbm,
                 src_v, dst_v, xr_v, m_v, agg_sh,
                 semA0, semA1, semA2, semA3, semB0, semB1,
                 semD0, semD1, semD2, semD3):
    cid = lax.axis_index("c")
    sid = lax.axis_index("s")
    wid = sid * NC + cid
    cbase = wid * CHUNKS_LO + jnp.minimum(wid, CHUNKS_REM)
    trips = CHUNKS_LO + jnp.where(wid < CHUNKS_REM, 1, 0)

    def zero_body(r):
        for jj in range(D // 16):
            m_v[0, r, pl.ds(jj * 16, 16)] = jnp.zeros((16,), jnp.float32)

    pl.loop(0, ZROWS)(zero_body)
    row0 = sid * ROWS_PER_TILE
    for k in range(ROWS_PER_TILE // ZROWS):
        pltpu.sync_copy(m_v.at[0], agg_sh.at[pl.ds(row0 + k * ZROWS, ZROWS)])
    if ZTAIL:
        pltpu.sync_copy(
            m_v.at[0, pl.ds(0, ZTAIL)],
            agg_sh.at[pl.ds(row0 + (ROWS_PER_TILE // ZROWS) * ZROWS, ZTAIL)])
    plsc.subcore_barrier()

    semA = [semA0, semA1, semA2, semA3]
    semB = [semB0, semB1]
    semD = [semD0, semD1, semD2, semD3]

    def issue_A(j, b):
        ebase = (cbase + j) * CHUNK
        pltpu.async_copy(src_hbm.at[pl.ds(ebase, CHUNK)], src_v.at[b],
                         semA[b])
        pltpu.async_copy(dst_hbm.at[pl.ds(ebase, CHUNK)], dst_v.at[b],
                         semA[b])
        pltpu.async_copy(ea_hbm.at[pl.ds(ebase, CHUNK)], m_v.at[b], semA[b])

    def wait_A(b):
        pltpu.make_async_copy(src_hbm.at[pl.ds(0, CHUNK)], src_v.at[b],
                              semA[b]).wait()
        pltpu.make_async_copy(dst_hbm.at[pl.ds(0, CHUNK)], dst_v.at[b],
                              semA[b]).wait()
        pltpu.make_async_copy(ea_hbm.at[pl.ds(0, CHUNK)], m_v.at[b],
                              semA[b]).wait()

    def issue_B(b, bx):
        pltpu.async_copy(x_hbm.at[src_v.at[b]], xr_v.at[bx], semB[bx])

    def wait_B(b, bx):
        pltpu.make_async_copy(x_hbm.at[src_v.at[b]], xr_v.at[bx],
                              semB[bx]).wait()

    def issue_D(b):
        pltpu.async_copy(m_v.at[b], agg_sh.at[dst_v.at[b]], semD[b],
                         add=True)

    def wait_D(b):
        pltpu.make_async_copy(m_v.at[b], agg_sh.at[dst_v.at[b]],
                              semD[b]).wait()

    def chunk_step(j, b):
        bx = b % 2
        nb1 = (b + 1) % 4

        @pl.when(j < trips)
        def _():
            @pl.when(j + 1 < trips)
            def _():
                wait_A(nb1)
                issue_B(nb1, 1 - bx)

            @pl.when(j + 2 < trips)
            def _():
                @pl.when(j >= 2)
                def _():
                    wait_D((b + 2) % 4)
                issue_A(j + 2, (b + 2) % 4)

            wait_B(b, bx)

            def row_body(r):
                for jj in range(D // 16):
                    sl = pl.ds(jj * 16, 16)
                    m_v[b, r, sl] = jnp.maximum(
                        m_v[b, r, sl] + xr_v[bx, r, sl], 0.0)

            pl.loop(0, CHUNK)(row_body)
            issue_D(b)

    issue_A(0, 0)
    issue_A(1, 1)
    wait_A(0)
    issue_B(0, 0)

    def quad_body(j):
        for u in range(4):
            chunk_step(j + u, u)

    pl.loop(0, trips, step=4)(quad_body)

    for b in range(4):
        wait_D(b)

    plsc.subcore_barrier()

    pltpu.sync_copy(agg_sh.at[pl.ds(row0, ROWS_PER_TILE)],
                    out_hbm.at[cid, pl.ds(row0, ROWS_PER_TILE)])


_sc_msg = functools.partial(
    pl.kernel,
    out_type=jax.ShapeDtypeStruct((NC, NP, D), jnp.float32),
    mesh=plsc.VectorSubcoreMesh(core_axis_name="c", subcore_axis_name="s"),
    scratch_types=[
        pltpu.VMEM((4, CHUNK), jnp.int32),
        pltpu.VMEM((4, CHUNK), jnp.int32),
        pltpu.VMEM((2, CHUNK, D), jnp.float32),
        pltpu.VMEM((4, CHUNK, D), jnp.float32),
        pltpu.VMEM_SHARED((NP, D), jnp.float32),
    ] + [pltpu.SemaphoreType.DMA] * 10,
)(_sc_msg_body)


EB = 16000


def _enc_body(eattr_ref, w0_ref, b0_ref, w1_ref, b1_ref, ea0_ref, ea1_ref):
    ea0 = eattr_ref[...] @ w0_ref[...] + b0_ref[...]
    ea0_ref[...] = ea0
    ea1_ref[...] = ea0 @ w1_ref[...] + b1_ref[...]


def _edge_encoders(edge_attr, enc0_W, enc0_b, enc1_W, enc1_b):
    return pl.pallas_call(
        _enc_body,
        grid=(E // EB,),
        in_specs=[
            pl.BlockSpec((EB, DE), lambda i: (i, 0)),
            pl.BlockSpec((DE, D), lambda i: (0, 0)),
            pl.BlockSpec((1, D), lambda i: (0, 0)),
            pl.BlockSpec((D, D), lambda i: (0, 0)),
            pl.BlockSpec((1, D), lambda i: (0, 0)),
        ],
        out_specs=[
            pl.BlockSpec((EB, D), lambda i: (i, 0)),
            pl.BlockSpec((EB, D), lambda i: (i, 0)),
        ],
        out_shape=[
            jax.ShapeDtypeStruct((E, D), jnp.float32),
            jax.ShapeDtypeStruct((E, D), jnp.float32),
        ],
    )(edge_attr, enc0_W, enc0_b.reshape(1, D), enc1_W, enc1_b.reshape(1, D))


NB = 5000


def _mlp0_body(x_ref, agg_ref, w1_ref, b1_ref, w2_ref, b2_ref, rw_ref,
               x1_ref, res_ref):
    x = x_ref[...]
    h = x + agg_ref[0] + agg_ref[1]
    t = jnp.maximum(h @ w1_ref[...] + b1_ref[...], 0.0)
    res = x @ rw_ref[...]
    res_ref[...] = res
    x1_ref[...] = jnp.maximum(t @ w2_ref[...] + b2_ref[...] + res, 0.0)


def _node_mlp0(x, agg, W1, b1, W2, b2, res_W):
    grid = N // NB
    return pl.pallas_call(
        _mlp0_body,
        grid=(grid,),
        in_specs=[
            pl.BlockSpec((NB, D), lambda i: (i, 0)),
            pl.BlockSpec((NC, NB, D), lambda i: (0, i, 0)),
            pl.BlockSpec((D, D), lambda i: (0, 0)),
            pl.BlockSpec((1, D), lambda i: (0, 0)),
            pl.BlockSpec((D, D), lambda i: (0, 0)),
            pl.BlockSpec((1, D), lambda i: (0, 0)),
            pl.BlockSpec((D, D), lambda i: (0, 0)),
        ],
        out_specs=[
            pl.BlockSpec((NB, D), lambda i: (i, 0)),
            pl.BlockSpec((NB, D), lambda i: (i, 0)),
        ],
        out_shape=[
            jax.ShapeDtypeStruct((N, D), jnp.float32),
            jax.ShapeDtypeStruct((N, D), jnp.float32),
        ],
    )(x, agg, W1, b1.reshape(1, D), W2, b2.reshape(1, D), res_W)


def _mlp1_body(x_ref, agg_ref, w1_ref, b1_ref, w2_ref, b2_ref, res_ref,
               x2_ref):
    h = x_ref[...] + agg_ref[0] + agg_ref[1]
    t = jnp.maximum(h @ w1_ref[...] + b1_ref[...], 0.0)
    x2_ref[...] = jnp.maximum(t @ w2_ref[...] + b2_ref[...] + res_ref[...], 0.0)


def _node_mlp1(x1, agg, W1, b1, W2, b2, res):
    grid = N // NB
    return pl.pallas_call(
        _mlp1_body,
        grid=(grid,),
        in_specs=[
            pl.BlockSpec((NB, D), lambda i: (i, 0)),
            pl.BlockSpec((NC, NB, D), lambda i: (0, i, 0)),
            pl.BlockSpec((D, D), lambda i: (0, 0)),
            pl.BlockSpec((1, D), lambda i: (0, 0)),
            pl.BlockSpec((D, D), lambda i: (0, 0)),
            pl.BlockSpec((1, D), lambda i: (0, 0)),
            pl.BlockSpec((NB, D), lambda i: (i, 0)),
        ],
        out_specs=pl.BlockSpec((NB, D), lambda i: (i, 0)),
        out_shape=jax.ShapeDtypeStruct((N, D), jnp.float32),
    )(x1, agg, W1, b1.reshape(1, D), W2, b2.reshape(1, D), res)


def _pool_body(x2_ref, batch_ref, cw_ref, cb_ref, out_ref):
    batch = batch_ref[...]
    gids = lax.broadcasted_iota(jnp.int32, (G, N), 0)
    oh = jnp.where(batch == gids, 1.0, 0.0)
    sums = oh @ x2_ref[...]
    cnts = jnp.sum(oh, axis=1, keepdims=True)
    pooled = sums / jnp.maximum(cnts, 1.0)
    out_ref[...] = pooled @ cw_ref[...] + cb_ref[...]


def _pool_cls(x2, batch, cls_W, cls_b):
    return pl.pallas_call(
        _pool_body,
        out_shape=jax.ShapeDtypeStruct((G, 2), jnp.float32),
    )(x2, batch.reshape(1, N), cls_W, cls_b.reshape(1, 2))


def kernel(x, edge_index, edge_attr, batch, enc0_W, enc0_b, enc1_W, enc1_b,
           gin0_W1, gin0_b1, gin0_W2, gin0_b2, gin1_W1, gin1_b1, gin1_W2, gin1_b2,
           res_W, cls_W, cls_b):
    src = edge_index[0]
    dst = edge_index[1]
    ea0, ea1 = _edge_encoders(edge_attr, enc0_W, enc0_b, enc1_W, enc1_b)
    agg0 = _sc_msg(x, ea0, src, dst)
    x1, res = _node_mlp0(x, agg0, gin0_W1, gin0_b1, gin0_W2, gin0_b2, res_W)

    agg1 = _sc_msg(x1, ea1, src, dst)
    x2 = _node_mlp1(x1, agg1, gin1_W1, gin1_b1, gin1_W2, gin1_b2, res)

    return _pool_cls(x2, batch, cls_W, cls_b)


# speedup vs baseline: 1.0945x; 1.0945x over previous
import functools

import jax
import jax.numpy as jnp
from jax import lax
from jax.experimental import pallas as pl
from jax.experimental.pallas import tpu as pltpu
from jax.experimental.pallas import tpu_sc as plsc

N = 10000
E = 320000
D = 128
DE = 16
G = 16

NC = 2
NS = 16
NW = NC * NS
CHUNK = 64
TOTAL_CHUNKS = E // CHUNK
CHUNKS_LO = TOTAL_CHUNKS // NW
CHUNKS_REM = TOTAL_CHUNKS % NW
NP = 10112
ROWS_PER_TILE = NP // NS
ZROWS = CHUNK
ZTAIL = ROWS_PER_TILE - (ROWS_PER_TILE // ZROWS) * ZROWS


def _sc_msg_body(x_hbm, ea_hbm, src_hbm, dst_hbm, out_hbm,
                 src_v, dst_v, xr_v, m_v, agg_sh,
                 semA0, semA1, semA2, semA3, semB0, semB1,
                 semD0, semD1, semD2, semD3):
    cid = lax.axis_index("c")
    sid = lax.axis_index("s")
    wid = sid * NC + cid
    cbase = wid * CHUNKS_LO + jnp.minimum(wid, CHUNKS_REM)
    trips = CHUNKS_LO + jnp.where(wid < CHUNKS_REM, 1, 0)

    def zero_body(r):
        for jj in range(D // 16):
            m_v[0, r, pl.ds(jj * 16, 16)] = jnp.zeros((16,), jnp.float32)

    pl.loop(0, ZROWS)(zero_body)
    row0 = sid * ROWS_PER_TILE
    for k in range(ROWS_PER_TILE // ZROWS):
        pltpu.sync_copy(m_v.at[0], agg_sh.at[pl.ds(row0 + k * ZROWS, ZROWS)])
    if ZTAIL:
        pltpu.sync_copy(
            m_v.at[0, pl.ds(0, ZTAIL)],
            agg_sh.at[pl.ds(row0 + (ROWS_PER_TILE // ZROWS) * ZROWS, ZTAIL)])
    plsc.subcore_barrier()

    semA = [semA0, semA1, semA2, semA3]
    semB = [semB0, semB1]
    semD = [semD0, semD1, semD2, semD3]

    def issue_A(j, b):
        ebase = (cbase + j) * CHUNK
        pltpu.async_copy(src_hbm.at[pl.ds(ebase, CHUNK)], src_v.at[b],
                         semA[b])
        pltpu.async_copy(dst_hbm.at[pl.ds(ebase, CHUNK)], dst_v.at[b],
                         semA[b])
        pltpu.async_copy(ea_hbm.at[pl.ds(ebase, CHUNK)], m_v.at[b], semA[b])

    def wait_A(b):
        pltpu.make_async_copy(src_hbm.at[pl.ds(0, CHUNK)], src_v.at[b],
                              semA[b]).wait()
        pltpu.make_async_copy(dst_hbm.at[pl.ds(0, CHUNK)], dst_v.at[b],
                              semA[b]).wait()
        pltpu.make_async_copy(ea_hbm.at[pl.ds(0, CHUNK)], m_v.at[b],
                              semA[b]).wait()

    def issue_B(b, bx):
        pltpu.async_copy(x_hbm.at[src_v.at[b]], xr_v.at[bx], semB[bx])

    def wait_B(b, bx):
        pltpu.make_async_copy(x_hbm.at[src_v.at[b]], xr_v.at[bx],
                              semB[bx]).wait()

    def issue_D(b):
        pltpu.async_copy(m_v.at[b], agg_sh.at[dst_v.at[b]], semD[b],
                         add=True)

    def wait_D(b):
        pltpu.make_async_copy(m_v.at[b], agg_sh.at[dst_v.at[b]],
                              semD[b]).wait()

    def chunk_step(j, b):
        bx = b % 2
        nb1 = (b + 1) % 4

        @pl.when(j < trips)
        def _():
            @pl.when(j + 2 < trips)
            def _():
                @pl.when(j >= 2)
                def _():
                    wait_D((b + 2) % 4)
                issue_A(j + 2, (b + 2) % 4)

            @pl.when(j + 1 < trips)
            def _():
                wait_A(nb1)
                issue_B(nb1, 1 - bx)

            wait_B(b, bx)

            def row_body(r):
                for jj in range(D // 16):
                    sl = pl.ds(jj * 16, 16)
                    m_v[b, r, sl] = jnp.maximum(
                        m_v[b, r, sl] + xr_v[bx, r, sl], 0.0)

            pl.loop(0, CHUNK)(row_body)
            issue_D(b)

    issue_A(0, 0)
    issue_A(1, 1)
    wait_A(0)
    issue_B(0, 0)

    def quad_body(j):
        for u in range(4):
            chunk_step(j + u, u)

    pl.loop(0, trips, step=4)(quad_body)

    for b in range(4):
        wait_D(b)

    plsc.subcore_barrier()

    pltpu.sync_copy(agg_sh.at[pl.ds(row0, ROWS_PER_TILE)],
                    out_hbm.at[cid, pl.ds(row0, ROWS_PER_TILE)])


_sc_msg = functools.partial(
    pl.kernel,
    out_type=jax.ShapeDtypeStruct((NC, NP, D), jnp.float32),
    mesh=plsc.VectorSubcoreMesh(core_axis_name="c", subcore_axis_name="s"),
    scratch_types=[
        pltpu.VMEM((4, CHUNK), jnp.int32),
        pltpu.VMEM((4, CHUNK), jnp.int32),
        pltpu.VMEM((2, CHUNK, D), jnp.float32),
        pltpu.VMEM((4, CHUNK, D), jnp.float32),
        pltpu.VMEM_SHARED((NP, D), jnp.float32),
    ] + [pltpu.SemaphoreType.DMA] * 10,
)(_sc_msg_body)


EB = 16000


def _enc_body(eattr_ref, w0_ref, b0_ref, w1_ref, b1_ref, ea0_ref, ea1_ref):
    ea0 = eattr_ref[...] @ w0_ref[...] + b0_ref[...]
    ea0_ref[...] = ea0
    ea1_ref[...] = ea0 @ w1_ref[...] + b1_ref[...]


def _edge_encoders(edge_attr, enc0_W, enc0_b, enc1_W, enc1_b):
    return pl.pallas_call(
        _enc_body,
        grid=(E // EB,),
        in_specs=[
            pl.BlockSpec((EB, DE), lambda i: (i, 0)),
            pl.BlockSpec((DE, D), lambda i: (0, 0)),
            pl.BlockSpec((1, D), lambda i: (0, 0)),
            pl.BlockSpec((D, D), lambda i: (0, 0)),
            pl.BlockSpec((1, D), lambda i: (0, 0)),
        ],
        out_specs=[
            pl.BlockSpec((EB, D), lambda i: (i, 0)),
            pl.BlockSpec((EB, D), lambda i: (i, 0)),
        ],
        out_shape=[
            jax.ShapeDtypeStruct((E, D), jnp.float32),
            jax.ShapeDtypeStruct((E, D), jnp.float32),
        ],
    )(edge_attr, enc0_W, enc0_b.reshape(1, D), enc1_W, enc1_b.reshape(1, D))


NB = 5000


def _mlp0_body(x_ref, agg_ref, w1_ref, b1_ref, w2_ref, b2_ref, rw_ref,
               x1_ref, res_ref):
    x = x_ref[...]
    h = x + agg_ref[0] + agg_ref[1]
    t = jnp.maximum(h @ w1_ref[...] + b1_ref[...], 0.0)
    res = x @ rw_ref[...]
    res_ref[...] = res
    x1_ref[...] = jnp.maximum(t @ w2_ref[...] + b2_ref[...] + res, 0.0)


def _node_mlp0(x, agg, W1, b1, W2, b2, res_W):
    grid = N // NB
    return pl.pallas_call(
        _mlp0_body,
        grid=(grid,),
        in_specs=[
            pl.BlockSpec((NB, D), lambda i: (i, 0)),
            pl.BlockSpec((NC, NB, D), lambda i: (0, i, 0)),
            pl.BlockSpec((D, D), lambda i: (0, 0)),
            pl.BlockSpec((1, D), lambda i: (0, 0)),
            pl.BlockSpec((D, D), lambda i: (0, 0)),
            pl.BlockSpec((1, D), lambda i: (0, 0)),
            pl.BlockSpec((D, D), lambda i: (0, 0)),
        ],
        out_specs=[
            pl.BlockSpec((NB, D), lambda i: (i, 0)),
            pl.BlockSpec((NB, D), lambda i: (i, 0)),
        ],
        out_shape=[
            jax.ShapeDtypeStruct((N, D), jnp.float32),
            jax.ShapeDtypeStruct((N, D), jnp.float32),
        ],
    )(x, agg, W1, b1.reshape(1, D), W2, b2.reshape(1, D), res_W)


def _mlp1_body(x_ref, agg_ref, w1_ref, b1_ref, w2_ref, b2_ref, res_ref,
               x2_ref):
    h = x_ref[...] + agg_ref[0] + agg_ref[1]
    t = jnp.maximum(h @ w1_ref[...] + b1_ref[...], 0.0)
    x2_ref[...] = jnp.maximum(t @ w2_ref[...] + b2_ref[...] + res_ref[...], 0.0)


def _node_mlp1(x1, agg, W1, b1, W2, b2, res):
    grid = N // NB
    return pl.pallas_call(
        _mlp1_body,
        grid=(grid,),
        in_specs=[
            pl.BlockSpec((NB, D), lambda i: (i, 0)),
            pl.BlockSpec((NC, NB, D), lambda i: (0, i, 0)),
            pl.BlockSpec((D, D), lambda i: (0, 0)),
            pl.BlockSpec((1, D), lambda i: (0, 0)),
            pl.BlockSpec((D, D), lambda i: (0, 0)),
            pl.BlockSpec((1, D), lambda i: (0, 0)),
            pl.BlockSpec((NB, D), lambda i: (i, 0)),
        ],
        out_specs=pl.BlockSpec((NB, D), lambda i: (i, 0)),
        out_shape=jax.ShapeDtypeStruct((N, D), jnp.float32),
    )(x1, agg, W1, b1.reshape(1, D), W2, b2.reshape(1, D), res)


def _pool_body(x2_ref, batch_ref, cw_ref, cb_ref, out_ref):
    batch = batch_ref[...]
    gids = lax.broadcasted_iota(jnp.int32, (G, N), 0)
    oh = jnp.where(batch == gids, 1.0, 0.0)
    sums = oh @ x2_ref[...]
    cnts = jnp.sum(oh, axis=1, keepdims=True)
    pooled = sums / jnp.maximum(cnts, 1.0)
    out_ref[...] = pooled @ cw_ref[...] + cb_ref[...]


def _pool_cls(x2, batch, cls_W, cls_b):
    return pl.pallas_call(
        _pool_body,
        out_shape=jax.ShapeDtypeStruct((G, 2), jnp.float32),
    )(x2, batch.reshape(1, N), cls_W, cls_b.reshape(1, 2))


def kernel(x, edge_index, edge_attr, batch, enc0_W, enc0_b, enc1_W, enc1_b,
           gin0_W1, gin0_b1, gin0_W2, gin0_b2, gin1_W1, gin1_b1, gin1_W2, gin1_b2,
           res_W, cls_W, cls_b):
    src = edge_index[0]
    dst = edge_index[1]
    ea0, ea1 = _edge_encoders(edge_attr, enc0_W, enc0_b, enc1_W, enc1_b)
    agg0 = _sc_msg(x, ea0, src, dst)
    x1, res = _node_mlp0(x, agg0, gin0_W1, gin0_b1, gin0_W2, gin0_b2, res_W)

    agg1 = _sc_msg(x1, ea1, src, dst)
    x2 = _node_mlp1(x1, agg1, gin1_W1, gin1_b1, gin1_W2, gin1_b2, res)

    return _pool_cls(x2, batch, cls_W, cls_b)


# speedup vs baseline: 1.0984x; 1.0035x over previous
import functools

import jax
import jax.numpy as jnp
from jax import lax
from jax.experimental import pallas as pl
from jax.experimental.pallas import tpu as pltpu
from jax.experimental.pallas import tpu_sc as plsc

N = 10000
E = 320000
D = 128
DE = 16
G = 16

NC = 2
NS = 16
NW = NC * NS
CHUNK = 64
TOTAL_CHUNKS = E // CHUNK
CHUNKS_LO = TOTAL_CHUNKS // NW
CHUNKS_REM = TOTAL_CHUNKS % NW
NP = 10112
ROWS_PER_TILE = NP // NS
ZROWS = CHUNK
ZTAIL = ROWS_PER_TILE - (ROWS_PER_TILE // ZROWS) * ZROWS


def _sc_msg_body(x_hbm, ea_hbm, src_hbm, dst_hbm, out_hbm,
                 src_v, dst_v, xr_v, m_v, agg_sh,
                 semA0, semA1, semA2, semA3, semB0, semB1, semB20, semB21,
                 semD0, semD1, semD2, semD3):
    cid = lax.axis_index("c")
    sid = lax.axis_index("s")
    wid = sid * NC + cid
    cbase = wid * CHUNKS_LO + jnp.minimum(wid, CHUNKS_REM)
    trips = CHUNKS_LO + jnp.where(wid < CHUNKS_REM, 1, 0)

    def zero_body(r):
        for jj in range(D // 16):
            m_v[0, r, pl.ds(jj * 16, 16)] = jnp.zeros((16,), jnp.float32)

    pl.loop(0, ZROWS)(zero_body)
    row0 = sid * ROWS_PER_TILE
    for k in range(ROWS_PER_TILE // ZROWS):
        pltpu.sync_copy(m_v.at[0], agg_sh.at[pl.ds(row0 + k * ZROWS, ZROWS)])
    if ZTAIL:
        pltpu.sync_copy(
            m_v.at[0, pl.ds(0, ZTAIL)],
            agg_sh.at[pl.ds(row0 + (ROWS_PER_TILE // ZROWS) * ZROWS, ZTAIL)])
    plsc.subcore_barrier()

    semA = [semA0, semA1, semA2, semA3]
    semB = [semB0, semB1]
    semB2 = [semB20, semB21]
    semD = [semD0, semD1, semD2, semD3]

    def issue_A(j, b):
        ebase = (cbase + j) * CHUNK
        pltpu.async_copy(src_hbm.at[pl.ds(ebase, CHUNK)], src_v.at[b],
                         semA[b])
        pltpu.async_copy(dst_hbm.at[pl.ds(ebase, CHUNK)], dst_v.at[b],
                         semA[b])
        pltpu.async_copy(ea_hbm.at[pl.ds(ebase, CHUNK)], m_v.at[b], semA[b])

    def wait_A(b):
        pltpu.make_async_copy(src_hbm.at[pl.ds(0, CHUNK)], src_v.at[b],
                              semA[b]).wait()
        pltpu.make_async_copy(dst_hbm.at[pl.ds(0, CHUNK)], dst_v.at[b],
                              semA[b]).wait()
        pltpu.make_async_copy(ea_hbm.at[pl.ds(0, CHUNK)], m_v.at[b],
                              semA[b]).wait()

    HC = CHUNK // 2

    def issue_B(b, bx):
        pltpu.async_copy(x_hbm.at[src_v.at[b, pl.ds(0, HC)]],
                         xr_v.at[bx, pl.ds(0, HC)], semB[bx])
        pltpu.async_copy(x_hbm.at[src_v.at[b, pl.ds(HC, HC)]],
                         xr_v.at[bx, pl.ds(HC, HC)], semB2[bx])

    def wait_B1(b, bx):
        pltpu.make_async_copy(x_hbm.at[src_v.at[b, pl.ds(0, HC)]],
                              xr_v.at[bx, pl.ds(0, HC)], semB[bx]).wait()

    def wait_B2(b, bx):
        pltpu.make_async_copy(x_hbm.at[src_v.at[b, pl.ds(HC, HC)]],
                              xr_v.at[bx, pl.ds(HC, HC)], semB2[bx]).wait()

    def issue_D(b):
        pltpu.async_copy(m_v.at[b], agg_sh.at[dst_v.at[b]], semD[b],
                         add=True)

    def wait_D(b):
        pltpu.make_async_copy(m_v.at[b], agg_sh.at[dst_v.at[b]],
                              semD[b]).wait()

    def chunk_step(j, b):
        bx = b % 2
        nb1 = (b + 1) % 4

        @pl.when(j < trips)
        def _():
            @pl.when(j + 2 < trips)
            def _():
                @pl.when(j >= 2)
                def _():
                    wait_D((b + 2) % 4)
                issue_A(j + 2, (b + 2) % 4)

            @pl.when(j + 1 < trips)
            def _():
                wait_A(nb1)
                issue_B(nb1, 1 - bx)

            def row_body(r):
                for jj in range(D // 16):
                    sl = pl.ds(jj * 16, 16)
                    m_v[b, r, sl] = jnp.maximum(
                        m_v[b, r, sl] + xr_v[bx, r, sl], 0.0)

            wait_B1(b, bx)
            pl.loop(0, HC)(row_body)
            wait_B2(b, bx)
            pl.loop(HC, CHUNK)(row_body)
            issue_D(b)

    issue_A(0, 0)
    issue_A(1, 1)
    wait_A(0)
    issue_B(0, 0)

    def quad_body(j):
        for u in range(4):
            chunk_step(j + u, u)

    pl.loop(0, trips, step=4)(quad_body)

    for b in range(4):
        wait_D(b)

    plsc.subcore_barrier()

    pltpu.sync_copy(agg_sh.at[pl.ds(row0, ROWS_PER_TILE)],
                    out_hbm.at[cid, pl.ds(row0, ROWS_PER_TILE)])


_sc_msg = functools.partial(
    pl.kernel,
    out_type=jax.ShapeDtypeStruct((NC, NP, D), jnp.float32),
    mesh=plsc.VectorSubcoreMesh(core_axis_name="c", subcore_axis_name="s"),
    scratch_types=[
        pltpu.VMEM((4, CHUNK), jnp.int32),
        pltpu.VMEM((4, CHUNK), jnp.int32),
        pltpu.VMEM((2, CHUNK, D), jnp.float32),
        pltpu.VMEM((4, CHUNK, D), jnp.float32),
        pltpu.VMEM_SHARED((NP, D), jnp.float32),
    ] + [pltpu.SemaphoreType.DMA] * 12,
)(_sc_msg_body)


EB = 16000


def _enc_body(eattr_ref, w0_ref, b0_ref, w1_ref, b1_ref, ea0_ref, ea1_ref):
    ea0 = eattr_ref[...] @ w0_ref[...] + b0_ref[...]
    ea0_ref[...] = ea0
    ea1_ref[...] = ea0 @ w1_ref[...] + b1_ref[...]


def _edge_encoders(edge_attr, enc0_W, enc0_b, enc1_W, enc1_b):
    return pl.pallas_call(
        _enc_body,
        grid=(E // EB,),
        in_specs=[
            pl.BlockSpec((EB, DE), lambda i: (i, 0)),
            pl.BlockSpec((DE, D), lambda i: (0, 0)),
            pl.BlockSpec((1, D), lambda i: (0, 0)),
            pl.BlockSpec((D, D), lambda i: (0, 0)),
            pl.BlockSpec((1, D), lambda i: (0, 0)),
        ],
        out_specs=[
            pl.BlockSpec((EB, D), lambda i: (i, 0)),
            pl.BlockSpec((EB, D), lambda i: (i, 0)),
        ],
        out_shape=[
            jax.ShapeDtypeStruct((E, D), jnp.float32),
            jax.ShapeDtypeStruct((E, D), jnp.float32),
        ],
    )(edge_attr, enc0_W, enc0_b.reshape(1, D), enc1_W, enc1_b.reshape(1, D))


NB = 5000


def _mlp0_body(x_ref, agg_ref, w1_ref, b1_ref, w2_ref, b2_ref, rw_ref,
               x1_ref, res_ref):
    x = x_ref[...]
    h = x + agg_ref[0] + agg_ref[1]
    t = jnp.maximum(h @ w1_ref[...] + b1_ref[...], 0.0)
    res = x @ rw_ref[...]
    res_ref[...] = res
    x1_ref[...] = jnp.maximum(t @ w2_ref[...] + b2_ref[...] + res, 0.0)


def _node_mlp0(x, agg, W1, b1, W2, b2, res_W):
    grid = N // NB
    return pl.pallas_call(
        _mlp0_body,
        grid=(grid,),
        in_specs=[
            pl.BlockSpec((NB, D), lambda i: (i, 0)),
            pl.BlockSpec((NC, NB, D), lambda i: (0, i, 0)),
            pl.BlockSpec((D, D), lambda i: (0, 0)),
            pl.BlockSpec((1, D), lambda i: (0, 0)),
            pl.BlockSpec((D, D), lambda i: (0, 0)),
            pl.BlockSpec((1, D), lambda i: (0, 0)),
            pl.BlockSpec((D, D), lambda i: (0, 0)),
        ],
        out_specs=[
            pl.BlockSpec((NB, D), lambda i: (i, 0)),
            pl.BlockSpec((NB, D), lambda i: (i, 0)),
        ],
        out_shape=[
            jax.ShapeDtypeStruct((N, D), jnp.float32),
            jax.ShapeDtypeStruct((N, D), jnp.float32),
        ],
    )(x, agg, W1, b1.reshape(1, D), W2, b2.reshape(1, D), res_W)


def _mlp1_body(x_ref, agg_ref, w1_ref, b1_ref, w2_ref, b2_ref, res_ref,
               x2_ref):
    h = x_ref[...] + agg_ref[0] + agg_ref[1]
    t = jnp.maximum(h @ w1_ref[...] + b1_ref[...], 0.0)
    x2_ref[...] = jnp.maximum(t @ w2_ref[...] + b2_ref[...] + res_ref[...], 0.0)


def _node_mlp1(x1, agg, W1, b1, W2, b2, res):
    grid = N // NB
    return pl.pallas_call(
        _mlp1_body,
        grid=(grid,),
        in_specs=[
            pl.BlockSpec((NB, D), lambda i: (i, 0)),
            pl.BlockSpec((NC, NB, D), lambda i: (0, i, 0)),
            pl.BlockSpec((D, D), lambda i: (0, 0)),
            pl.BlockSpec((1, D), lambda i: (0, 0)),
            pl.BlockSpec((D, D), lambda i: (0, 0)),
            pl.BlockSpec((1, D), lambda i: (0, 0)),
            pl.BlockSpec((NB, D), lambda i: (i, 0)),
        ],
        out_specs=pl.BlockSpec((NB, D), lambda i: (i, 0)),
        out_shape=jax.ShapeDtypeStruct((N, D), jnp.float32),
    )(x1, agg, W1, b1.reshape(1, D), W2, b2.reshape(1, D), res)


def _pool_body(x2_ref, batch_ref, cw_ref, cb_ref, out_ref):
    batch = batch_ref[...]
    gids = lax.broadcasted_iota(jnp.int32, (G, N), 0)
    oh = jnp.where(batch == gids, 1.0, 0.0)
    sums = oh @ x2_ref[...]
    cnts = jnp.sum(oh, axis=1, keepdims=True)
    pooled = sums / jnp.maximum(cnts, 1.0)
    out_ref[...] = pooled @ cw_ref[...] + cb_ref[...]


def _pool_cls(x2, batch, cls_W, cls_b):
    return pl.pallas_call(
        _pool_body,
        out_shape=jax.ShapeDtypeStruct((G, 2), jnp.float32),
    )(x2, batch.reshape(1, N), cls_W, cls_b.reshape(1, 2))


def kernel(x, edge_index, edge_attr, batch, enc0_W, enc0_b, enc1_W, enc1_b,
           gin0_W1, gin0_b1, gin0_W2, gin0_b2, gin1_W1, gin1_b1, gin1_W2, gin1_b2,
           res_W, cls_W, cls_b):
    src = edge_index[0]
    dst = edge_index[1]
    ea0, ea1 = _edge_encoders(edge_attr, enc0_W, enc0_b, enc1_W, enc1_b)
    agg0 = _sc_msg(x, ea0, src, dst)
    x1, res = _node_mlp0(x, agg0, gin0_W1, gin0_b1, gin0_W2, gin0_b2, res_W)

    agg1 = _sc_msg(x1, ea1, src, dst)
    x2 = _node_mlp1(x1, agg1, gin1_W1, gin1_b1, gin1_W2, gin1_b2, res)

    return _pool_cls(x2, batch, cls_W, cls_b)
